# Initial kernel scaffold; baseline (speedup 1.0000x reference)
import functools

import jax
import jax.numpy as jnp
from jax import lax
from jax.experimental import pallas as pl
from jax.experimental.pallas import tpu as pltpu
from jax.experimental.pallas import tpu_sc as plsc

N = 10000
E = 320000
D = 128

NC = 2
NS = 16
NW = NC * NS
EW = E // NW
C = 100
NCHUNK = EW // C

ROWS_PER_TILE = N // NS


def _sc_body(x_hbm, src_hbm, dst_hbm, out_hbm,
             src_v, dst_v, rows0, rows1, sem0, sem1, acc):
    c = lax.axis_index("c")
    s = lax.axis_index("s")
    wid = s * NC + c

    pltpu.sync_copy(src_hbm.at[wid], src_v)
    pltpu.sync_copy(dst_hbm.at[wid], dst_v)

    @functools.partial(lax.fori_loop, 0, C * 8, init_val=None)
    def _(t, _):
        rows0[t // 8, pl.ds((t % 8) * 16, 16)] = jnp.zeros((16,), jnp.float32)
        return None

    tbase = s * ROWS_PER_TILE

    @functools.partial(lax.fori_loop, 0, 6, init_val=None)
    def _(r, _):
        pltpu.sync_copy(rows0, acc.at[pl.ds(tbase + r * C, C)])
        return None

    pltpu.sync_copy(rows0.at[pl.ds(0, 25)], acc.at[pl.ds(tbase + 6 * C, 25)])

    plsc.subcore_barrier()

    pltpu.async_copy(x_hbm.at[src_v.at[0]], rows0, sem0)

    @functools.partial(lax.fori_loop, 0, NCHUNK // 2, init_val=None)
    def _(i, _):
        j = i * 2
        pltpu.async_copy(x_hbm.at[src_v.at[j + 1]], rows1, sem1)
        pltpu.make_async_copy(x_hbm.at[src_v.at[j]], rows0, sem0).wait()
        pltpu.sync_copy(rows0, acc.at[dst_v.at[j]], add=True)

        @pl.when(j + 2 < NCHUNK)
        def _():
            pltpu.async_copy(x_hbm.at[src_v.at[j + 2]], rows0, sem0)

        pltpu.make_async_copy(x_hbm.at[src_v.at[j + 1]], rows1, sem1).wait()
        pltpu.sync_copy(rows1, acc.at[dst_v.at[j + 1]], add=True)
        return None

    plsc.subcore_barrier()

    @functools.partial(lax.fori_loop, 0, 6, init_val=None)
    def _(r, _):
        pltpu.sync_copy(acc.at[pl.ds(tbase + r * C, C)], rows0)
        pltpu.sync_copy(rows0, out_hbm.at[c].at[pl.ds(tbase + r * C, C)])
        return None

    pltpu.sync_copy(acc.at[pl.ds(tbase + 6 * C, 25)], rows0.at[pl.ds(0, 25)])
    pltpu.sync_copy(rows0.at[pl.ds(0, 25)], out_hbm.at[c].at[pl.ds(tbase + 6 * C, 25)])


_sc_segment_sum = functools.partial(
    pl.kernel,
    out_type=jax.ShapeDtypeStruct((NC, N, D), jnp.float32),
    mesh=plsc.VectorSubcoreMesh(
        core_axis_name="c", subcore_axis_name="s",
        num_cores=NC, num_subcores=NS),
    scratch_types=[
        pltpu.VMEM((NCHUNK, C), jnp.int32),
        pltpu.VMEM((NCHUNK, C), jnp.int32),
        pltpu.VMEM((C, D), jnp.float32),
        pltpu.VMEM((C, D), jnp.float32),
        pltpu.SemaphoreType.DMA,
        pltpu.SemaphoreType.DMA,
        pltpu.VMEM_SHARED((N, D), jnp.float32),
    ],
)(_sc_body)


BLK = 1000


def _mlp_body(relu_out, h_ref, p0_ref, p1_ref, wa_ref, ba_ref, wb_ref, bb_ref,
              o_ref):
    z = h_ref[...] + p0_ref[...] + p1_ref[...]
    a = jnp.dot(z, wa_ref[...], preferred_element_type=jnp.float32) + ba_ref[...]
    a = jnp.where(a > 0, a, a * 0.01)
    o = jnp.dot(a, wb_ref[...], preferred_element_type=jnp.float32) + bb_ref[...]
    if relu_out:
        o = jnp.where(o > 0, o, o * 0.01)
    o_ref[...] = o


def _mlp_tc(h, p0, p1, wa_t, ba, wb_t, bb, relu_out):
    row_spec = pl.BlockSpec((BLK, D), lambda i: (i, 0))
    full_spec = pl.BlockSpec((D, D), lambda i: (0, 0))
    bias_spec = pl.BlockSpec((1, D), lambda i: (0, 0))
    return pl.pallas_call(
        functools.partial(_mlp_body, relu_out),
        grid=(N // BLK,),
        in_specs=[row_spec, row_spec, row_spec,
                  full_spec, bias_spec, full_spec, bias_spec],
        out_specs=row_spec,
        out_shape=jax.ShapeDtypeStruct((N, D), jnp.float32),
    )(h, p0, p1, wa_t, ba.reshape(1, D), wb_t, bb.reshape(1, D))


def kernel(x, edge_index, W1a, b1a, W1b, b1b, W2a, b2a, W2b, b2b,
           W3a, b3a, W3b, b3b):
    src = edge_index[0].reshape(NW, NCHUNK, C)
    dst = edge_index[1].reshape(NW, NCHUNK, C)

    w3b_t = jnp.zeros((D, D), jnp.float32).at[:, :2].set(W3b.T)
    b3b_p = jnp.zeros((D,), jnp.float32).at[:2].set(b3b)

    p = _sc_segment_sum(x, src, dst)
    h = _mlp_tc(x, p[0], p[1], W1a.T, b1a, W1b.T, b1b, relu_out=True)

    p = _sc_segment_sum(h, src, dst)
    h = _mlp_tc(h, p[0], p[1], W2a.T, b2a, W2b.T, b2b, relu_out=True)

    p = _sc_segment_sum(h, src, dst)
    out = _mlp_tc(h, p[0], p[1], W3a.T, b3a, w3b_t, b3b_p, relu_out=False)

    return out[:, :2]


# speedup vs baseline: 7.7921x; 7.7921x over previous
import functools

import jax
import jax.numpy as jnp
from jax import lax
from jax.experimental import pallas as pl
from jax.experimental.pallas import tpu as pltpu
from jax.experimental.pallas import tpu_sc as plsc

N = 10000
E = 320000
D = 128

NC = 2
NS = 16
NW = NC * NS
EW = E // NW
C = 50
NCHUNK = EW // C
IB = 20
NIB = NCHUNK // IB

NPAD = 10240
RT = NPAD // NS
WC = 40


def _sc_body(x_hbm, src_hbm, dst_hbm, out_hbm,
             src_v, dst_v, rows0, rows1, sem0, sem1, acc):
    c = lax.axis_index("c")
    s = lax.axis_index("s")
    wid = s * NC + c

    @functools.partial(lax.fori_loop, 0, C * 8, init_val=None)
    def _(t, _):
        rows0[t // 8, pl.ds((t % 8) * 16, 16)] = jnp.zeros((16,), jnp.float32)
        return None

    tbase = s * RT
    zsrc = rows0.at[pl.ds(0, WC)]

    @functools.partial(lax.fori_loop, 0, RT // WC, init_val=None)
    def _(r, _):
        pltpu.sync_copy(zsrc, acc.at[pl.ds(tbase + r * WC, WC)])
        return None

    plsc.subcore_barrier()

    @functools.partial(lax.fori_loop, 0, NIB, init_val=None)
    def _(b, _):
        pltpu.sync_copy(src_hbm.at[wid, b], src_v)
        pltpu.sync_copy(dst_hbm.at[wid, b], dst_v)
        pltpu.async_copy(x_hbm.at[src_v.at[0]], rows0, sem0)

        @functools.partial(lax.fori_loop, 0, IB // 2, init_val=None)
        def _(i, _):
            j = i * 2
            pltpu.async_copy(x_hbm.at[src_v.at[j + 1]], rows1, sem1)
            pltpu.make_async_copy(x_hbm.at[src_v.at[j]], rows0, sem0).wait()
            pltpu.sync_copy(rows0, acc.at[dst_v.at[j]], add=True)

            @pl.when(j + 2 < IB)
            def _():
                pltpu.async_copy(x_hbm.at[src_v.at[j + 2]], rows0, sem0)

            pltpu.make_async_copy(x_hbm.at[src_v.at[j + 1]], rows1, sem1).wait()
            pltpu.sync_copy(rows1, acc.at[dst_v.at[j + 1]], add=True)
            return None

        return None

    plsc.subcore_barrier()

    @functools.partial(lax.fori_loop, 0, RT // WC, init_val=None)
    def _(r, _):
        pltpu.sync_copy(acc.at[pl.ds(tbase + r * WC, WC)], zsrc)
        pltpu.sync_copy(zsrc, out_hbm.at[c].at[pl.ds(tbase + r * WC, WC)])
        return None


_sc_segment_sum = functools.partial(
    pl.kernel,
    out_type=jax.ShapeDtypeStruct((NC, NPAD, D), jnp.float32),
    mesh=plsc.VectorSubcoreMesh(
        core_axis_name="c", subcore_axis_name="s",
        num_cores=NC, num_subcores=NS),
    scratch_types=[
        pltpu.VMEM((IB, C), jnp.int32),
        pltpu.VMEM((IB, C), jnp.int32),
        pltpu.VMEM((C, D), jnp.float32),
        pltpu.VMEM((C, D), jnp.float32),
        pltpu.SemaphoreType.DMA,
        pltpu.SemaphoreType.DMA,
        pltpu.VMEM_SHARED((NPAD, D), jnp.float32),
    ],
)(_sc_body)


BLK = 1000


def _mlp_body(relu_out, h_ref, p0_ref, p1_ref, wa_ref, ba_ref, wb_ref, bb_ref,
              o_ref):
    z = h_ref[...] + p0_ref[...] + p1_ref[...]
    a = jnp.dot(z, wa_ref[...], preferred_element_type=jnp.float32) + ba_ref[...]
    a = jnp.where(a > 0, a, a * 0.01)
    o = jnp.dot(a, wb_ref[...], preferred_element_type=jnp.float32) + bb_ref[...]
    if relu_out:
        o = jnp.where(o > 0, o, o * 0.01)
    o_ref[...] = o


def _mlp_tc(h, p, wa_t, ba, wb_t, bb, relu_out):
    row_spec = pl.BlockSpec((BLK, D), lambda i: (i, 0))
    part_spec = pl.BlockSpec((1, BLK, D), lambda i: (0, i, 0))
    full_spec = pl.BlockSpec((D, D), lambda i: (0, 0))
    bias_spec = pl.BlockSpec((1, D), lambda i: (0, 0))
    p0 = p[0:1]
    p1 = p[1:2]
    body = functools.partial(_mlp_body, relu_out)

    def wrapped(h_ref, p0_ref, p1_ref, wa_ref, ba_ref, wb_ref, bb_ref, o_ref):
        body(h_ref, p0_ref.at[0], p1_ref.at[0], wa_ref, ba_ref, wb_ref,
             bb_ref, o_ref)

    return pl.pallas_call(
        wrapped,
        grid=(N // BLK,),
        in_specs=[row_spec, part_spec, part_spec,
                  full_spec, bias_spec, full_spec, bias_spec],
        out_specs=row_spec,
        out_shape=jax.ShapeDtypeStruct((N, D), jnp.float32),
    )(h, p0, p1, wa_t, ba.reshape(1, D), wb_t, bb.reshape(1, D))


def kernel(x, edge_index, W1a, b1a, W1b, b1b, W2a, b2a, W2b, b2b,
           W3a, b3a, W3b, b3b):
    src = edge_index[0].reshape(NW, NIB, IB, C)
    dst = edge_index[1].reshape(NW, NIB, IB, C)

    w3b_t = jnp.zeros((D, D), jnp.float32).at[:, :2].set(W3b.T)
    b3b_p = jnp.zeros((D,), jnp.float32).at[:2].set(b3b)

    p = _sc_segment_sum(x, src, dst)
    h = _mlp_tc(x, p, W1a.T, b1a, W1b.T, b1b, relu_out=True)

    p = _sc_segment_sum(h, src, dst)
    h = _mlp_tc(h, p, W2a.T, b2a, W2b.T, b2b, relu_out=True)

    p = _sc_segment_sum(h, src, dst)
    out = _mlp_tc(h, p, W3a.T, b3a, w3b_t, b3b_p, relu_out=False)

    return out[:, :2]
